# baseline (speedup 1.0000x reference)
import functools

import jax
import jax.numpy as jnp
from jax import lax
from jax.experimental import pallas as pl
from jax.experimental.pallas import tpu as pltpu
from jax.experimental.pallas import tpu_sc as plsc

DIM = 1024
SEQ = 8192
BATCH = 2

NS = 4
SSEQ = SEQ // NS

NUM_WORKERS = 32
ROWS_PER_W = SSEQ // NUM_WORKERS
CHUNK = 32
NCHUNK = ROWS_PER_W // CHUNK

BS = 256
NBLK = SSEQ // BS


def _sc_gather_body(sbase, pe_hbm, pos_hbm, g_hbm,
                    idx_v, buf0, buf1, sem0, sem1):
    wid = lax.axis_index("s") * 2 + lax.axis_index("c")
    base = wid * ROWS_PER_W
    pltpu.sync_copy(pos_hbm.at[pl.ds(sbase + base, ROWS_PER_W)], idx_v)
    bufs = (buf0, buf1)
    sems = (sem0, sem1)
    copies = []
    for c in range(NCHUNK):
        copies.append(
            pltpu.async_copy(
                pe_hbm.at[idx_v.at[pl.ds(c * CHUNK, CHUNK)]],
                bufs[c % 2],
                sems[c % 2],
            )
        )
        if c > 0:
            copies[c - 1].wait()
            pltpu.sync_copy(
                bufs[(c - 1) % 2],
                g_hbm.at[pl.ds(base + (c - 1) * CHUNK, CHUNK)],
            )
    copies[NCHUNK - 1].wait()
    pltpu.sync_copy(
        bufs[(NCHUNK - 1) % 2],
        g_hbm.at[pl.ds(base + (NCHUNK - 1) * CHUNK, CHUNK)],
    )


def _make_gather(i):
    return functools.partial(
        pl.kernel,
        out_type=jax.ShapeDtypeStruct((SSEQ, DIM), jnp.float32),
        mesh=plsc.VectorSubcoreMesh(core_axis_name="c", subcore_axis_name="s"),
        scratch_types=[
            pltpu.VMEM((ROWS_PER_W,), jnp.int32),
            pltpu.VMEM((CHUNK, DIM), jnp.float32),
            pltpu.VMEM((CHUNK, DIM), jnp.float32),
            pltpu.SemaphoreType.DMA,
            pltpu.SemaphoreType.DMA,
        ],
    )(functools.partial(_sc_gather_body, i * SSEQ))


_gathers = [_make_gather(i) for i in range(NS)]


def _tc_add_first(q_ref, k_ref, g_ref, oq_ref, ok_ref):
    g = g_ref[...][None, :, :]
    oq_ref[...] = q_ref[...] + g
    ok_ref[...] = k_ref[...] + g


def _tc_add_chained(q_ref, k_ref, g_ref, oqp_ref, okp_ref, oq_ref, ok_ref):
    del oqp_ref, okp_ref
    g = g_ref[...][None, :, :]
    oq_ref[...] = q_ref[...] + g
    ok_ref[...] = k_ref[...] + g


_OUT_SHAPE = [
    jax.ShapeDtypeStruct((BATCH, SEQ, DIM), jnp.float32),
    jax.ShapeDtypeStruct((BATCH, SEQ, DIM), jnp.float32),
]


def _make_add(i):
    off = i * NBLK
    qk_spec = pl.BlockSpec((BATCH, BS, DIM), lambda j, off=off: (0, j + off, 0))
    g_spec = pl.BlockSpec((BS, DIM), lambda j: (j, 0))
    out_specs = [
        pl.BlockSpec((BATCH, BS, DIM), lambda j, off=off: (0, j + off, 0)),
        pl.BlockSpec((BATCH, BS, DIM), lambda j, off=off: (0, j + off, 0)),
    ]
    if i == 0:
        return pl.pallas_call(
            _tc_add_first,
            grid=(NBLK,),
            in_specs=[qk_spec, qk_spec, g_spec],
            out_specs=out_specs,
            out_shape=_OUT_SHAPE,
        )
    any_spec = pl.BlockSpec(memory_space=pl.ANY)
    return pl.pallas_call(
        _tc_add_chained,
        grid=(NBLK,),
        in_specs=[qk_spec, qk_spec, g_spec, any_spec, any_spec],
        out_specs=out_specs,
        out_shape=_OUT_SHAPE,
        input_output_aliases={3: 0, 4: 1},
    )


_adds = [_make_add(i) for i in range(NS)]


@jax.jit
def kernel(q, k, pos, pe):
    gs = [_gathers[i](pe, pos) for i in range(NS)]
    oq, ok = _adds[0](q, k, gs[0])
    for i in range(1, NS):
        oq, ok = _adds[i](q, k, gs[i], oq, ok)
    return oq, ok


# speedup vs baseline: 1.0136x; 1.0136x over previous
import functools

import jax
import jax.numpy as jnp
from jax import lax
from jax.experimental import pallas as pl
from jax.experimental.pallas import tpu as pltpu
from jax.experimental.pallas import tpu_sc as plsc

DIM = 1024
SEQ = 8192
BATCH = 2

NUM_WORKERS = 32
ROWS_PER_W = SEQ // NUM_WORKERS
CHUNK = 32
NCHUNK = ROWS_PER_W // CHUNK

BS = 256
NBLK = SEQ // BS


def _sc_gather_body(pe_hbm, pos_hbm, g_hbm, idx_v, buf0, buf1, sem0, sem1):
    wid = lax.axis_index("s") * 2 + lax.axis_index("c")
    base = wid * ROWS_PER_W
    pltpu.sync_copy(pos_hbm.at[pl.ds(base, ROWS_PER_W)], idx_v)
    bufs = (buf0, buf1)
    sems = (sem0, sem1)
    copies = []
    for c in range(NCHUNK):
        copies.append(
            pltpu.async_copy(
                pe_hbm.at[idx_v.at[pl.ds(c * CHUNK, CHUNK)]],
                bufs[c % 2],
                sems[c % 2],
            )
        )
        if c > 0:
            copies[c - 1].wait()
            pltpu.sync_copy(
                bufs[(c - 1) % 2],
                g_hbm.at[pl.ds(base + (c - 1) * CHUNK, CHUNK)],
            )
    copies[NCHUNK - 1].wait()
    pltpu.sync_copy(
        bufs[(NCHUNK - 1) % 2],
        g_hbm.at[pl.ds(base + (NCHUNK - 1) * CHUNK, CHUNK)],
    )


_gather = functools.partial(
    pl.kernel,
    out_type=jax.ShapeDtypeStruct((SEQ, DIM), jnp.float32),
    mesh=plsc.VectorSubcoreMesh(core_axis_name="c", subcore_axis_name="s"),
    scratch_types=[
        pltpu.VMEM((ROWS_PER_W,), jnp.int32),
        pltpu.VMEM((CHUNK, DIM), jnp.float32),
        pltpu.VMEM((CHUNK, DIM), jnp.float32),
        pltpu.SemaphoreType.DMA,
        pltpu.SemaphoreType.DMA,
    ],
)(_sc_gather_body)


def _tc_add(q_ref, k_ref, g_ref, oq_ref, ok_ref):
    g = g_ref[...][None, :, :]
    oq_ref[...] = q_ref[...] + g
    ok_ref[...] = k_ref[...] + g


_fused_add = pl.pallas_call(
    _tc_add,
    grid=(NBLK,),
    in_specs=[
        pl.BlockSpec((BATCH, BS, DIM), lambda j: (0, j, 0)),
        pl.BlockSpec((BATCH, BS, DIM), lambda j: (0, j, 0)),
        pl.BlockSpec((BS, DIM), lambda j: (j, 0)),
    ],
    out_specs=[
        pl.BlockSpec((BATCH, BS, DIM), lambda j: (0, j, 0)),
        pl.BlockSpec((BATCH, BS, DIM), lambda j: (0, j, 0)),
    ],
    out_shape=[
        jax.ShapeDtypeStruct((BATCH, SEQ, DIM), jnp.float32),
        jax.ShapeDtypeStruct((BATCH, SEQ, DIM), jnp.float32),
    ],
)


@jax.jit
def kernel(q, k, pos, pe):
    g = _gather(pe, pos)
    oq, ok = _fused_add(q, k, g)
    return oq, ok
